# Initial kernel scaffold; baseline (speedup 1.0000x reference)
import jax
import jax.numpy as jnp
from jax.experimental import pallas as pl


def kernel(snn_batch, x, edge_index, batch, params):
    raise NotImplementedError("write your pallas kernel here")


# speedup vs baseline: 4.4450x; 4.4450x over previous
import functools

import jax
import jax.numpy as jnp
from jax import lax
from jax.experimental import pallas as pl
from jax.experimental.pallas import tpu as pltpu
from jax.experimental.pallas import tpu_sc as plsc

N_NODES = 10000
D = 128
E = 320000
N_GRAPHS = 128
N_LAYERS = 7
BETA = 0.85

NC, NS, L = 2, 16, 16
NW = NC * NS
CHUNK = 128
CPT = 79
PER_TILE = CPT * CHUNK
E_PAD = NW * PER_TILE
AGG_ROWS = 10112
STRIPE = AGG_ROWS // NS
SINK = N_NODES

_HI = jax.lax.Precision.HIGHEST


def _sc_body(h_hbm, src_hbm, dst_hbm, out0, out1, agg_sh, src_v, dst_v, rows_v, sem):
    core = lax.axis_index("c")
    sub = lax.axis_index("s")
    wid = core * NS + sub

    def _zrow(i, carry):
        for c in range(D // L):
            rows_v[i, pl.ds(c * L, L)] = jnp.zeros((L,), jnp.float32)
        return carry

    lax.fori_loop(0, CHUNK, _zrow, 0)
    base = sub * STRIPE
    for k in range(STRIPE // CHUNK):
        pltpu.sync_copy(rows_v, agg_sh.at[pl.ds(base + k * CHUNK, CHUNK)])
    rem = STRIPE % CHUNK
    if rem:
        pltpu.sync_copy(
            rows_v.at[pl.ds(0, rem)],
            agg_sh.at[pl.ds(base + (STRIPE // CHUNK) * CHUNK, rem)],
        )
    plsc.subcore_barrier()

    pltpu.sync_copy(src_hbm.at[wid], src_v)
    pltpu.sync_copy(dst_hbm.at[wid], dst_v)

    def _ebody(j, carry):
        pltpu.async_copy(h_hbm.at[src_v.at[j]], rows_v, sem).wait()
        pltpu.sync_copy(rows_v, agg_sh.at[dst_v.at[j]], add=True)
        return carry

    lax.fori_loop(0, CPT, _ebody, 0)
    plsc.subcore_barrier()

    @pl.when(core == 0)
    def _():
        pltpu.sync_copy(agg_sh.at[pl.ds(base, STRIPE)], out0.at[pl.ds(base, STRIPE)])

    @pl.when(core == 1)
    def _():
        pltpu.sync_copy(agg_sh.at[pl.ds(base, STRIPE)], out1.at[pl.ds(base, STRIPE)])


_sc_pass = pl.kernel(
    _sc_body,
    out_type=(
        jax.ShapeDtypeStruct((AGG_ROWS, D), jnp.float32),
        jax.ShapeDtypeStruct((AGG_ROWS, D), jnp.float32),
    ),
    mesh=plsc.VectorSubcoreMesh(
        core_axis_name="c", subcore_axis_name="s", num_cores=NC, num_subcores=NS
    ),
    scratch_types=[
        pltpu.VMEM_SHARED((AGG_ROWS, D), jnp.float32),
        pltpu.VMEM((CPT, CHUNK), jnp.int32),
        pltpu.VMEM((CPT, CHUNK), jnp.int32),
        pltpu.VMEM((CHUNK, D), jnp.float32),
        pltpu.SemaphoreType.DMA,
    ],
)


def _layer_body(a0, a1, h, wrel, wroot, brel, out):
    agg = a0[...] + a1[...]
    out[...] = jnp.maximum(
        jnp.dot(agg, wrel[...], preferred_element_type=jnp.float32, precision=_HI)
        + jnp.dot(h[...], wroot[...], preferred_element_type=jnp.float32, precision=_HI)
        + brel[...],
        0.0,
    )


_BLK = 1000


def _tc_layer(a0, a1, h, wrel, wroot, brel):
    return pl.pallas_call(
        _layer_body,
        grid=(N_NODES // _BLK,),
        in_specs=[
            pl.BlockSpec((_BLK, D), lambda i: (i, 0)),
            pl.BlockSpec((_BLK, D), lambda i: (i, 0)),
            pl.BlockSpec((_BLK, D), lambda i: (i, 0)),
            pl.BlockSpec((D, D), lambda i: (0, 0)),
            pl.BlockSpec((D, D), lambda i: (0, 0)),
            pl.BlockSpec((1, D), lambda i: (0, 0)),
        ],
        out_specs=pl.BlockSpec((_BLK, D), lambda i: (i, 0)),
        out_shape=jax.ShapeDtypeStruct((N_NODES, D), jnp.float32),
    )(a0, a1, h, wrel, wroot, brel)


def _head_body(h, batch2, snn, w1, b1, w2, b2, linw, linb, fw1, fw2, fb, out):
    memb = (
        batch2[...] == lax.broadcasted_iota(jnp.int32, (N_NODES, N_GRAPHS), 1)
    ).astype(jnp.float32)
    sums = lax.dot_general(
        memb, h[...], (((0,), (0,)), ((), ())),
        preferred_element_type=jnp.float32, precision=_HI,
    )
    counts = jnp.sum(memb, axis=0)
    pooled = sums / jnp.maximum(counts, 1.0)[:, None]
    gnn = (
        jnp.dot(pooled, linw[...], preferred_element_type=jnp.float32, precision=_HI)
        + linb[...]
    )
    hh = jnp.maximum(
        jnp.dot(snn[...], w1[...], preferred_element_type=jnp.float32, precision=_HI)
        + b1[...],
        0.0,
    )
    snl = BETA * (
        jnp.dot(hh, w2[...], preferred_element_type=jnp.float32, precision=_HI)
        + b2[...]
    )
    out[...] = (
        jnp.dot(snl, fw1[...], preferred_element_type=jnp.float32, precision=_HI)
        + jnp.dot(gnn, fw2[...], preferred_element_type=jnp.float32, precision=_HI)
        + fb[...]
    )


def _tc_head(h, batch2, snn, w1, b1, w2, b2, linw, linb, fw1, fw2, fb):
    return pl.pallas_call(
        _head_body,
        out_shape=jax.ShapeDtypeStruct((N_GRAPHS, N_GRAPHS), jnp.float32),
    )(h, batch2, snn, w1, b1, w2, b2, linw, linb, fw1, fw2, fb)


def kernel(snn_batch, x, edge_index, batch, params):
    src = edge_index[0]
    dst = edge_index[1]
    pad = E_PAD - E
    src_p = jnp.concatenate([src, jnp.zeros((pad,), jnp.int32)]).reshape(NW, CPT, CHUNK)
    dst_p = jnp.concatenate([dst, jnp.full((pad,), SINK, jnp.int32)]).reshape(
        NW, CPT, CHUNK
    )

    h = x
    for i in range(N_LAYERS):
        a0, a1 = _sc_pass(h, src_p, dst_p)
        h = _tc_layer(
            a0,
            a1,
            h,
            params["gnn_Wrel"][i],
            params["gnn_Wroot"][i],
            params["gnn_brel"][i].reshape(1, D),
        )

    return _tc_head(
        h,
        batch.reshape(N_NODES, 1),
        snn_batch,
        params["snn_W1"],
        params["snn_b1"].reshape(1, -1),
        params["snn_W2"],
        params["snn_b2"].reshape(1, -1),
        params["gnn_lin_W"],
        params["gnn_lin_b"].reshape(1, -1),
        params["fusion_W"][:N_GRAPHS],
        params["fusion_W"][N_GRAPHS:],
        params["fusion_b"].reshape(1, -1),
    )
